# Initial kernel scaffold; baseline (speedup 1.0000x reference)
import jax
import jax.numpy as jnp
from jax.experimental import pallas as pl


def kernel(x):
    raise NotImplementedError("write your pallas kernel here")


# speedup vs baseline: 30.3968x; 30.3968x over previous
import jax
import jax.numpy as jnp
import numpy as np
from jax.experimental import pallas as pl

_NUM_CLASSES = 80
_A = 5
_H = 32
_W = 32
_N = _A * _H * _W
_HW = _H * _W
_STRIDE = 16.0
_CONF_T = 0.5
_IOU_T = 0.45
_ANCHORS = np.array(
    [[1.3221, 1.73145], [3.19275, 4.00944], [5.05587, 8.09892],
     [9.47112, 4.84053], [11.2364, 10.0071]], dtype=np.float32)
_BLK = 512
_NB = _N // _BLK
_REC = 16


def _decode_kernel(x_ref, rec_ref, srow_ref):
    pos = jax.lax.broadcasted_iota(jnp.int32, (1, _HW), 1)
    gx = (pos % _W).astype(jnp.float32)
    gy = (pos // _W).astype(jnp.float32)
    for a in range(_A):
        base = a * (5 + _NUM_CLASSES)
        blk = x_ref[base:base + 5 + _NUM_CLASSES, :]
        cx = (jax.nn.sigmoid(blk[0:1, :]) + gx) * _STRIDE
        cy = (jax.nn.sigmoid(blk[1:2, :]) + gy) * _STRIDE
        w = jnp.exp(blk[2:3, :]) * float(_ANCHORS[a, 0]) * _STRIDE
        h = jnp.exp(blk[3:4, :]) * float(_ANCHORS[a, 1]) * _STRIDE
        conf = jax.nn.sigmoid(blk[4:5, :])
        pcls = jax.nn.sigmoid(blk[5:, :])
        best = pcls[0:1, :]
        bidx = jnp.zeros((1, _HW), jnp.float32)
        for c in range(1, _NUM_CLASSES):
            cur = pcls[c:c + 1, :]
            gt = cur > best
            best = jnp.where(gt, cur, best)
            bidx = jnp.where(gt, jnp.float32(c), bidx)
        x1 = cx - w / 2.0
        y1 = cy - h / 2.0
        x2 = cx + w / 2.0
        y2 = cy + h / 2.0
        area = jnp.maximum(x2 - x1, 0.0) * jnp.maximum(y2 - y1, 0.0)
        zero = jnp.zeros((5, _HW), jnp.float32)
        rows = jnp.concatenate(
            [cx, cy, w, h, conf, bidx, x1, y1, x2, y2, area, zero], axis=0)
        rec_ref[a * _HW:(a + 1) * _HW, :] = rows.T
        srow_ref[0:1, a * _HW:(a + 1) * _HW] = conf


def _rank_kernel(rec_ref, srow_ref, rank_ref):
    j = pl.program_id(0)
    s_col = rec_ref[:, 4:5]
    ridx = jax.lax.broadcasted_iota(jnp.int32, (_BLK, 1), 0) + j * _BLK
    acc = jnp.zeros((_BLK, 1), jnp.int32)
    for c in range(_NB):
        s_all = srow_ref[0:1, c * _BLK:(c + 1) * _BLK]
        cidx = jax.lax.broadcasted_iota(jnp.int32, (1, _BLK), 1) + c * _BLK
        ahead = (s_all > s_col) | ((s_all == s_col) & (cidx < ridx))
        acc = acc + jnp.sum(ahead.astype(jnp.int32), axis=1, keepdims=True)
    rank_ref[:, :] = acc


def _perm_kernel(rank_ref, rec_ref, out_ref):
    r = pl.program_id(0)
    tgt = jax.lax.broadcasted_iota(jnp.int32, (1, _BLK), 1) + r * _BLK
    acc = jnp.zeros((_BLK, _REC), jnp.float32)
    for c in range(_NB):
        rk = rank_ref[c * _BLK:(c + 1) * _BLK, :]
        onehot = (rk == tgt).astype(jnp.float32)
        rc = rec_ref[c * _BLK:(c + 1) * _BLK, :]
        acc = acc + jax.lax.dot_general(
            onehot, rc, (((0,), (0,)), ((), ())),
            precision=jax.lax.Precision.HIGHEST,
            preferred_element_type=jnp.float32)
    out_ref[:, :] = acc


def _trans_kernel(rec_ref, srT_ref, k0r_ref, k0c_ref):
    blk = rec_ref[:, :]
    t = blk.T
    srT_ref[:, :] = t
    k0r_ref[:, :] = (t[4:5, :] > _CONF_T).astype(jnp.int32)
    k0c_ref[:, :] = (blk[:, 4:5] > _CONF_T).astype(jnp.int32)


def _build_kernel(rec_ref, srT_ref, c_ref):
    jb = pl.program_id(0)
    ib = pl.program_id(1)

    @pl.when(jb > ib)
    def _zero():
        c_ref[:, :] = jnp.zeros((_BLK, _BLK), jnp.int8)

    @pl.when(jb <= ib)
    def _tile():
        x1c = rec_ref[:, 6:7]
        y1c = rec_ref[:, 7:8]
        x2c = rec_ref[:, 8:9]
        y2c = rec_ref[:, 9:10]
        ac = rec_ref[:, 10:11]
        sc = rec_ref[:, 4:5]
        x1r = srT_ref[6:7, :]
        y1r = srT_ref[7:8, :]
        x2r = srT_ref[8:9, :]
        y2r = srT_ref[9:10, :]
        ar = srT_ref[10:11, :]
        xx1 = jnp.maximum(x1c, x1r)
        yy1 = jnp.maximum(y1c, y1r)
        xx2 = jnp.minimum(x2c, x2r)
        yy2 = jnp.minimum(y2c, y2r)
        inter = jnp.maximum(xx2 - xx1, 0.0) * jnp.maximum(yy2 - yy1, 0.0)
        iou = inter / (ac + ar - inter + 1e-9)
        jidx = jax.lax.broadcasted_iota(jnp.int32, (_BLK, 1), 0) + jb * _BLK
        iidx = jax.lax.broadcasted_iota(jnp.int32, (1, _BLK), 1) + ib * _BLK
        cond = (iou > _IOU_T) & (jidx < iidx) & (sc > _CONF_T)
        c_ref[:, :] = cond.astype(jnp.int8)


def _sweep_kernel(c_ref, k0r_ref, kcol_ref, krow_out_ref, kcol_out_ref):
    supp = jnp.zeros((1, _BLK), jnp.int32)
    for c in range(_NB):
        mc = (c_ref[c * _BLK:(c + 1) * _BLK, :].astype(jnp.int32)
              * kcol_ref[c * _BLK:(c + 1) * _BLK, :])
        supp = jnp.maximum(supp, jnp.max(mc, axis=0, keepdims=True))
    knew = k0r_ref[:, :] * (1 - jnp.minimum(supp, 1))
    krow_out_ref[:, :] = knew
    kcol_out_ref[:, :] = knew.T


def _fin_kernel(rec_ref, kcol_ref, out_ref):
    k = kcol_ref[:, :].astype(jnp.float32)
    out_ref[:, :] = rec_ref[:, 0:6] * k


def kernel(x):
    X = x.reshape(_A * (5 + _NUM_CLASSES), _HW)

    rec, srow = pl.pallas_call(
        _decode_kernel,
        out_shape=[
            jax.ShapeDtypeStruct((_N, _REC), jnp.float32),
            jax.ShapeDtypeStruct((1, _N), jnp.float32),
        ],
    )(X)

    rank = pl.pallas_call(
        _rank_kernel,
        grid=(_NB,),
        in_specs=[
            pl.BlockSpec((_BLK, _REC), lambda j: (j, 0)),
            pl.BlockSpec((1, _N), lambda j: (0, 0)),
        ],
        out_specs=pl.BlockSpec((_BLK, 1), lambda j: (j, 0)),
        out_shape=jax.ShapeDtypeStruct((_N, 1), jnp.int32),
    )(rec, srow)

    srec = pl.pallas_call(
        _perm_kernel,
        grid=(_NB,),
        in_specs=[
            pl.BlockSpec((_N, 1), lambda r: (0, 0)),
            pl.BlockSpec((_N, _REC), lambda r: (0, 0)),
        ],
        out_specs=pl.BlockSpec((_BLK, _REC), lambda r: (r, 0)),
        out_shape=jax.ShapeDtypeStruct((_N, _REC), jnp.float32),
    )(rank, rec)

    srT, k0r, k0c = pl.pallas_call(
        _trans_kernel,
        grid=(_NB,),
        in_specs=[pl.BlockSpec((_BLK, _REC), lambda r: (r, 0))],
        out_specs=[
            pl.BlockSpec((_REC, _BLK), lambda r: (0, r)),
            pl.BlockSpec((1, _BLK), lambda r: (0, r)),
            pl.BlockSpec((_BLK, 1), lambda r: (r, 0)),
        ],
        out_shape=[
            jax.ShapeDtypeStruct((_REC, _N), jnp.float32),
            jax.ShapeDtypeStruct((1, _N), jnp.int32),
            jax.ShapeDtypeStruct((_N, 1), jnp.int32),
        ],
    )(srec)

    C = pl.pallas_call(
        _build_kernel,
        grid=(_NB, _NB),
        in_specs=[
            pl.BlockSpec((_BLK, _REC), lambda jb, ib: (jb, 0)),
            pl.BlockSpec((_REC, _BLK), lambda jb, ib: (0, ib)),
        ],
        out_specs=pl.BlockSpec((_BLK, _BLK), lambda jb, ib: (jb, ib)),
        out_shape=jax.ShapeDtypeStruct((_N, _N), jnp.int8),
    )(srec, srT)

    sweep = pl.pallas_call(
        _sweep_kernel,
        grid=(_NB,),
        in_specs=[
            pl.BlockSpec((_N, _BLK), lambda i: (0, i)),
            pl.BlockSpec((1, _BLK), lambda i: (0, i)),
            pl.BlockSpec((_N, 1), lambda i: (0, 0)),
        ],
        out_specs=[
            pl.BlockSpec((1, _BLK), lambda i: (0, i)),
            pl.BlockSpec((_BLK, 1), lambda i: (i, 0)),
        ],
        out_shape=[
            jax.ShapeDtypeStruct((1, _N), jnp.int32),
            jax.ShapeDtypeStruct((_N, 1), jnp.int32),
        ],
    )

    def body(carry):
        krow, kcol, _ = carry
        krow2, kcol2 = sweep(C, k0r, kcol)
        return krow2, kcol2, jnp.any(krow2 != krow)

    def cond(carry):
        return carry[2]

    krow, kcol, _ = jax.lax.while_loop(
        cond, body, (k0r, k0c, jnp.bool_(True)))

    out = pl.pallas_call(
        _fin_kernel,
        out_shape=jax.ShapeDtypeStruct((_N, 6), jnp.float32),
    )(srec, kcol)
    return out


# speedup vs baseline: 52.5293x; 1.7281x over previous
import jax
import jax.numpy as jnp
import numpy as np
from jax.experimental import pallas as pl
from jax.experimental.pallas import tpu as pltpu

_NUM_CLASSES = 80
_A = 5
_H = 32
_W = 32
_N = _A * _H * _W
_HW = _H * _W
_STRIDE = 16.0
_CONF_T = 0.5
_IOU_T = 0.45
_ANCHORS = np.array(
    [[1.3221, 1.73145], [3.19275, 4.00944], [5.05587, 8.09892],
     [9.47112, 4.84053], [11.2364, 10.0071]], dtype=np.float32)
_BLK = 512
_NB = _N // _BLK
_REC = 16


def _decode_kernel(x_ref, rec_ref, srow_ref):
    pos = jax.lax.broadcasted_iota(jnp.int32, (1, _HW), 1)
    gx = (pos % _W).astype(jnp.float32)
    gy = (pos // _W).astype(jnp.float32)
    for a in range(_A):
        base = a * (5 + _NUM_CLASSES)
        blk = x_ref[base:base + 5 + _NUM_CLASSES, :]
        cx = (jax.nn.sigmoid(blk[0:1, :]) + gx) * _STRIDE
        cy = (jax.nn.sigmoid(blk[1:2, :]) + gy) * _STRIDE
        w = jnp.exp(blk[2:3, :]) * float(_ANCHORS[a, 0]) * _STRIDE
        h = jnp.exp(blk[3:4, :]) * float(_ANCHORS[a, 1]) * _STRIDE
        conf = jax.nn.sigmoid(blk[4:5, :])
        pcls = jax.nn.sigmoid(blk[5:, :])
        best = pcls[0:1, :]
        bidx = jnp.zeros((1, _HW), jnp.float32)
        for c in range(1, _NUM_CLASSES):
            cur = pcls[c:c + 1, :]
            gt = cur > best
            best = jnp.where(gt, cur, best)
            bidx = jnp.where(gt, jnp.float32(c), bidx)
        x1 = cx - w / 2.0
        y1 = cy - h / 2.0
        x2 = cx + w / 2.0
        y2 = cy + h / 2.0
        area = jnp.maximum(x2 - x1, 0.0) * jnp.maximum(y2 - y1, 0.0)
        zero = jnp.zeros((5, _HW), jnp.float32)
        rows = jnp.concatenate(
            [cx, cy, w, h, conf, bidx, x1, y1, x2, y2, area, zero], axis=0)
        rec_ref[a * _HW:(a + 1) * _HW, :] = rows.T
        srow_ref[0:1, a * _HW:(a + 1) * _HW] = conf


def _rank_kernel(rec_ref, srow_ref, rank_ref):
    j = pl.program_id(0)
    s_col = rec_ref[:, 4:5]
    ridx = jax.lax.broadcasted_iota(jnp.int32, (_BLK, 1), 0) + j * _BLK
    acc = jnp.zeros((_BLK, 1), jnp.int32)
    for c in range(_NB):
        s_all = srow_ref[0:1, c * _BLK:(c + 1) * _BLK]
        cidx = jax.lax.broadcasted_iota(jnp.int32, (1, _BLK), 1) + c * _BLK
        ahead = (s_all > s_col) | ((s_all == s_col) & (cidx < ridx))
        acc = acc + jnp.sum(ahead.astype(jnp.int32), axis=1, keepdims=True)
    rank_ref[:, :] = acc


def _perm_kernel(rank_ref, rec_ref, out_ref):
    r = pl.program_id(0)
    tgt = jax.lax.broadcasted_iota(jnp.int32, (1, _BLK), 1) + r * _BLK
    acc = jnp.zeros((_BLK, _REC), jnp.float32)
    for c in range(_NB):
        rk = rank_ref[c * _BLK:(c + 1) * _BLK, :]
        onehot = (rk == tgt).astype(jnp.float32)
        rc = rec_ref[c * _BLK:(c + 1) * _BLK, :]
        acc = acc + jax.lax.dot_general(
            onehot, rc, (((0,), (0,)), ((), ())),
            precision=jax.lax.Precision.HIGHEST,
            preferred_element_type=jnp.float32)
    out_ref[:, :] = acc


def _trans_kernel(rec_ref, srT_ref):
    srT_ref[:, :] = rec_ref[:, :].T


def _iou_tile(rec_ref, x1r, y1r, x2r, y2r, ar):
    x1c = rec_ref[:, 6:7]
    y1c = rec_ref[:, 7:8]
    x2c = rec_ref[:, 8:9]
    y2c = rec_ref[:, 9:10]
    ac = rec_ref[:, 10:11]
    xx1 = jnp.maximum(x1c, x1r)
    yy1 = jnp.maximum(y1c, y1r)
    xx2 = jnp.minimum(x2c, x2r)
    yy2 = jnp.minimum(y2c, y2r)
    inter = jnp.maximum(xx2 - xx1, 0.0) * jnp.maximum(yy2 - yy1, 0.0)
    return inter / (ac + ar - inter + 1e-9)


def _nms_kernel(rec_ref, srT_ref, krow_ref, kcol_ref, keff_ref, kfin_ref):
    b = pl.program_id(0)
    c = pl.program_id(1)

    @pl.when((b == 0) & (c == 0))
    def _init():
        for cc in range(_NB):
            k0 = (srT_ref[4:5, cc * _BLK:(cc + 1) * _BLK]
                  > _CONF_T).astype(jnp.int32)
            keff_ref[8 * cc:8 * cc + 1, :] = k0

    @pl.when(c == b)
    def _diag():
        x1r = rec_ref[:, 6:7].T
        y1r = rec_ref[:, 7:8].T
        x2r = rec_ref[:, 8:9].T
        y2r = rec_ref[:, 9:10].T
        ar = rec_ref[:, 10:11].T
        iou = _iou_tile(rec_ref, x1r, y1r, x2r, y2r, ar)
        jl = jax.lax.broadcasted_iota(jnp.int32, (_BLK, 1), 0)
        il = jax.lax.broadcasted_iota(jnp.int32, (1, _BLK), 1)
        cbb = ((iou > _IOU_T) & (jl < il)).astype(jnp.int32)
        k0e = keff_ref[pl.ds(8 * b, 8), :][0:1, :]

        def cond(carry):
            return carry[1] > 0

        def body(carry):
            krow, _ = carry
            m = cbb * krow.T
            supp = jnp.max(m, axis=0, keepdims=True)
            knew = k0e * (1 - jnp.minimum(supp, 1))
            chg = jnp.max(jnp.abs(knew - krow))
            return knew, chg

        kfin, _ = jax.lax.while_loop(cond, body, (k0e, jnp.int32(1)))
        kfin_ref[:, :] = kfin.T
        krow_ref[:, :] = kfin
        kcol_ref[:, :] = kfin.T

    @pl.when(c > b)
    def _offdiag():
        x1r = srT_ref[6:7, pl.ds(c * _BLK, _BLK)]
        y1r = srT_ref[7:8, pl.ds(c * _BLK, _BLK)]
        x2r = srT_ref[8:9, pl.ds(c * _BLK, _BLK)]
        y2r = srT_ref[9:10, pl.ds(c * _BLK, _BLK)]
        ar = srT_ref[10:11, pl.ds(c * _BLK, _BLK)]
        iou = _iou_tile(rec_ref, x1r, y1r, x2r, y2r, ar)
        m = (iou > _IOU_T).astype(jnp.int32) * kfin_ref[:, :]
        supp = jnp.max(m, axis=0, keepdims=True)
        old8 = keff_ref[pl.ds(8 * c, 8), :]
        keff_ref[pl.ds(8 * c, 8), :] = old8 * (1 - jnp.minimum(supp, 1))


def _fin_kernel(rec_ref, kcol_ref, out_ref):
    k = kcol_ref[:, :].astype(jnp.float32)
    out_ref[:, :] = rec_ref[:, 0:6] * k


def kernel(x):
    X = x.reshape(_A * (5 + _NUM_CLASSES), _HW)

    rec, srow = pl.pallas_call(
        _decode_kernel,
        out_shape=[
            jax.ShapeDtypeStruct((_N, _REC), jnp.float32),
            jax.ShapeDtypeStruct((1, _N), jnp.float32),
        ],
    )(X)

    rank = pl.pallas_call(
        _rank_kernel,
        grid=(_NB,),
        in_specs=[
            pl.BlockSpec((_BLK, _REC), lambda j: (j, 0)),
            pl.BlockSpec((1, _N), lambda j: (0, 0)),
        ],
        out_specs=pl.BlockSpec((_BLK, 1), lambda j: (j, 0)),
        out_shape=jax.ShapeDtypeStruct((_N, 1), jnp.int32),
    )(rec, srow)

    srec = pl.pallas_call(
        _perm_kernel,
        grid=(_NB,),
        in_specs=[
            pl.BlockSpec((_N, 1), lambda r: (0, 0)),
            pl.BlockSpec((_N, _REC), lambda r: (0, 0)),
        ],
        out_specs=pl.BlockSpec((_BLK, _REC), lambda r: (r, 0)),
        out_shape=jax.ShapeDtypeStruct((_N, _REC), jnp.float32),
    )(rank, rec)

    srT = pl.pallas_call(
        _trans_kernel,
        grid=(_NB,),
        in_specs=[pl.BlockSpec((_BLK, _REC), lambda r: (r, 0))],
        out_specs=pl.BlockSpec((_REC, _BLK), lambda r: (0, r)),
        out_shape=jax.ShapeDtypeStruct((_REC, _N), jnp.float32),
    )(srec)

    krow, kcol = pl.pallas_call(
        _nms_kernel,
        grid=(_NB, _NB),
        in_specs=[
            pl.BlockSpec((_BLK, _REC), lambda b, c: (b, 0)),
            pl.BlockSpec((_REC, _N), lambda b, c: (0, 0)),
        ],
        out_specs=[
            pl.BlockSpec((1, _BLK), lambda b, c: (0, b)),
            pl.BlockSpec((_BLK, 1), lambda b, c: (b, 0)),
        ],
        out_shape=[
            jax.ShapeDtypeStruct((1, _N), jnp.int32),
            jax.ShapeDtypeStruct((_N, 1), jnp.int32),
        ],
        scratch_shapes=[
            pltpu.VMEM((8 * _NB, _BLK), jnp.int32),
            pltpu.VMEM((_BLK, 1), jnp.int32),
        ],
    )(srec, srT)

    out = pl.pallas_call(
        _fin_kernel,
        out_shape=jax.ShapeDtypeStruct((_N, 6), jnp.float32),
    )(srec, kcol)
    return out


# speedup vs baseline: 77.5582x; 1.4765x over previous
import functools

import jax
import jax.numpy as jnp
import numpy as np
from jax import lax
from jax.experimental import pallas as pl
from jax.experimental.pallas import tpu as pltpu
from jax.experimental.pallas import tpu_sc as plsc

_NUM_CLASSES = 80
_A = 5
_H = 32
_W = 32
_N = _A * _H * _W
_HW = _H * _W
_STRIDE = 16.0
_CONF_T = 0.5
_IOU_T = 0.45
_ANCHORS = np.array(
    [[1.3221, 1.73145], [3.19275, 4.00944], [5.05587, 8.09892],
     [9.47112, 4.84053], [11.2364, 10.0071]], dtype=np.float32)
_BLK = 512
_NB = _N // _BLK
_REC = 128


def _decode_kernel(x_ref, rec_ref, srow_ref):
    pos = jax.lax.broadcasted_iota(jnp.int32, (1, _HW), 1)
    gx = (pos % _W).astype(jnp.float32)
    gy = (pos // _W).astype(jnp.float32)
    for a in range(_A):
        base = a * (5 + _NUM_CLASSES)
        blk = x_ref[base:base + 5 + _NUM_CLASSES, :]
        cx = (jax.nn.sigmoid(blk[0:1, :]) + gx) * _STRIDE
        cy = (jax.nn.sigmoid(blk[1:2, :]) + gy) * _STRIDE
        w = jnp.exp(blk[2:3, :]) * float(_ANCHORS[a, 0]) * _STRIDE
        h = jnp.exp(blk[3:4, :]) * float(_ANCHORS[a, 1]) * _STRIDE
        conf = jax.nn.sigmoid(blk[4:5, :])
        pcls = jax.nn.sigmoid(blk[5:, :])
        best = pcls[0:1, :]
        bidx = jnp.zeros((1, _HW), jnp.float32)
        for c in range(1, _NUM_CLASSES):
            cur = pcls[c:c + 1, :]
            gt = cur > best
            best = jnp.where(gt, cur, best)
            bidx = jnp.where(gt, jnp.float32(c), bidx)
        x1 = cx - w / 2.0
        y1 = cy - h / 2.0
        x2 = cx + w / 2.0
        y2 = cy + h / 2.0
        area = jnp.maximum(x2 - x1, 0.0) * jnp.maximum(y2 - y1, 0.0)
        zero = jnp.zeros((_REC - 11, _HW), jnp.float32)
        rows = jnp.concatenate(
            [cx, cy, w, h, conf, bidx, x1, y1, x2, y2, area, zero], axis=0)
        rec_ref[a * _HW:(a + 1) * _HW, :] = rows.T
        srow_ref[0:1, a * _HW:(a + 1) * _HW] = conf


def _rank_kernel(rec_ref, srow_ref, rank_ref):
    j = pl.program_id(0)
    s_col = rec_ref[:, 4:5]
    ridx = jax.lax.broadcasted_iota(jnp.int32, (_BLK, 1), 0) + j * _BLK
    acc = jnp.zeros((_BLK, 1), jnp.int32)
    for c in range(_NB):
        s_all = srow_ref[0:1, c * _BLK:(c + 1) * _BLK]
        cidx = jax.lax.broadcasted_iota(jnp.int32, (1, _BLK), 1) + c * _BLK
        ahead = (s_all > s_col) | ((s_all == s_col) & (cidx < ridx))
        acc = acc + jnp.sum(ahead.astype(jnp.int32), axis=1, keepdims=True)
    rank_ref[:, :] = acc


_SC_WORKERS = 32
_SC_PER_W = _N // _SC_WORKERS
_SC_CHUNK = 80


def _sc_scatter_body(rank_hbm, rec_hbm, out_hbm, idx_v, rows_v, sem):
    wid = lax.axis_index("s") * 2 + lax.axis_index("c")
    base = wid * _SC_PER_W
    for t in range(_SC_PER_W // _SC_CHUNK):
        off = base + t * _SC_CHUNK
        pltpu.sync_copy(rank_hbm.at[pl.ds(off, _SC_CHUNK)], idx_v.at[t])
        pltpu.sync_copy(rec_hbm.at[pl.ds(off, _SC_CHUNK)], rows_v.at[t])
        pltpu.async_copy(rows_v.at[t], out_hbm.at[idx_v.at[t]], sem).wait()


def _sc_scatter(rank_flat, rec):
    mesh = plsc.VectorSubcoreMesh(core_axis_name="c", subcore_axis_name="s")
    f = functools.partial(
        pl.kernel,
        out_type=jax.ShapeDtypeStruct((_N, _REC), jnp.float32),
        mesh=mesh,
        scratch_types=[
            pltpu.VMEM((_SC_PER_W // _SC_CHUNK, _SC_CHUNK), jnp.int32),
            pltpu.VMEM((_SC_PER_W // _SC_CHUNK, _SC_CHUNK, _REC),
                       jnp.float32),
            pltpu.SemaphoreType.DMA,
        ],
    )(_sc_scatter_body)
    return f(rank_flat, rec)


def _perm_kernel(rank_ref, rec_ref, out_ref):
    r = pl.program_id(0)
    tgt = jax.lax.broadcasted_iota(jnp.int32, (1, _BLK), 1) + r * _BLK
    acc = jnp.zeros((_BLK, _REC), jnp.float32)
    for c in range(_NB):
        rk = rank_ref[c * _BLK:(c + 1) * _BLK, :]
        onehot = (rk == tgt).astype(jnp.float32)
        rc = rec_ref[c * _BLK:(c + 1) * _BLK, :]
        acc = acc + jax.lax.dot_general(
            onehot, rc, (((0,), (0,)), ((), ())),
            precision=jax.lax.Precision.HIGHEST,
            preferred_element_type=jnp.float32)
    out_ref[:, :] = acc


def _trans_kernel(rec_ref, srT_ref):
    srT_ref[:, :] = rec_ref[:, :].T


def _iou_tile(rec_ref, x1r, y1r, x2r, y2r, ar):
    x1c = rec_ref[:, 6:7]
    y1c = rec_ref[:, 7:8]
    x2c = rec_ref[:, 8:9]
    y2c = rec_ref[:, 9:10]
    ac = rec_ref[:, 10:11]
    xx1 = jnp.maximum(x1c, x1r)
    yy1 = jnp.maximum(y1c, y1r)
    xx2 = jnp.minimum(x2c, x2r)
    yy2 = jnp.minimum(y2c, y2r)
    inter = jnp.maximum(xx2 - xx1, 0.0) * jnp.maximum(yy2 - yy1, 0.0)
    return inter / (ac + ar - inter + 1e-9)


def _nms_kernel(rec_ref, srT_ref, krow_ref, kcol_ref, keff_ref, kfin_ref):
    b = pl.program_id(0)
    c = pl.program_id(1)

    @pl.when((b == 0) & (c == 0))
    def _init():
        for cc in range(_NB):
            k0 = (srT_ref[4:5, cc * _BLK:(cc + 1) * _BLK]
                  > _CONF_T).astype(jnp.int32)
            keff_ref[8 * cc:8 * cc + 1, :] = k0

    @pl.when(c == b)
    def _diag():
        x1r = rec_ref[:, 6:7].T
        y1r = rec_ref[:, 7:8].T
        x2r = rec_ref[:, 8:9].T
        y2r = rec_ref[:, 9:10].T
        ar = rec_ref[:, 10:11].T
        iou = _iou_tile(rec_ref, x1r, y1r, x2r, y2r, ar)
        jl = jax.lax.broadcasted_iota(jnp.int32, (_BLK, 1), 0)
        il = jax.lax.broadcasted_iota(jnp.int32, (1, _BLK), 1)
        cbb = ((iou > _IOU_T) & (jl < il)).astype(jnp.int32)
        k0e = keff_ref[pl.ds(8 * b, 8), :][0:1, :]

        def cond(carry):
            return carry[1] > 0

        def body(carry):
            krow, _ = carry
            m = cbb * krow.T
            supp = jnp.max(m, axis=0, keepdims=True)
            knew = k0e * (1 - jnp.minimum(supp, 1))
            chg = jnp.max(jnp.abs(knew - krow))
            return knew, chg

        kfin, _ = jax.lax.while_loop(cond, body, (k0e, jnp.int32(1)))
        kfin_ref[:, :] = kfin.T
        krow_ref[:, :] = kfin
        kcol_ref[:, :] = kfin.T

    @pl.when(c > b)
    def _offdiag():
        x1r = srT_ref[6:7, pl.ds(c * _BLK, _BLK)]
        y1r = srT_ref[7:8, pl.ds(c * _BLK, _BLK)]
        x2r = srT_ref[8:9, pl.ds(c * _BLK, _BLK)]
        y2r = srT_ref[9:10, pl.ds(c * _BLK, _BLK)]
        ar = srT_ref[10:11, pl.ds(c * _BLK, _BLK)]
        iou = _iou_tile(rec_ref, x1r, y1r, x2r, y2r, ar)
        m = (iou > _IOU_T).astype(jnp.int32) * kfin_ref[:, :]
        supp = jnp.max(m, axis=0, keepdims=True)
        old8 = keff_ref[pl.ds(8 * c, 8), :]
        keff_ref[pl.ds(8 * c, 8), :] = old8 * (1 - jnp.minimum(supp, 1))


def _fin_kernel(rec_ref, kcol_ref, out_ref):
    k = kcol_ref[:, :].astype(jnp.float32)
    out_ref[:, :] = rec_ref[:, 0:6] * k


def kernel(x):
    X = x.reshape(_A * (5 + _NUM_CLASSES), _HW)

    rec, srow = pl.pallas_call(
        _decode_kernel,
        out_shape=[
            jax.ShapeDtypeStruct((_N, _REC), jnp.float32),
            jax.ShapeDtypeStruct((1, _N), jnp.float32),
        ],
    )(X)

    rank = pl.pallas_call(
        _rank_kernel,
        grid=(_NB,),
        in_specs=[
            pl.BlockSpec((_BLK, _REC), lambda j: (j, 0)),
            pl.BlockSpec((1, _N), lambda j: (0, 0)),
        ],
        out_specs=pl.BlockSpec((_BLK, 1), lambda j: (j, 0)),
        out_shape=jax.ShapeDtypeStruct((_N, 1), jnp.int32),
    )(rec, srow)

    srec = _sc_scatter(rank.reshape(_N), rec)

    srT = pl.pallas_call(
        _trans_kernel,
        grid=(_NB,),
        in_specs=[pl.BlockSpec((_BLK, _REC), lambda r: (r, 0))],
        out_specs=pl.BlockSpec((_REC, _BLK), lambda r: (0, r)),
        out_shape=jax.ShapeDtypeStruct((_REC, _N), jnp.float32),
    )(srec)

    krow, kcol = pl.pallas_call(
        _nms_kernel,
        grid=(_NB, _NB),
        in_specs=[
            pl.BlockSpec((_BLK, _REC), lambda b, c: (b, 0)),
            pl.BlockSpec((_REC, _N), lambda b, c: (0, 0)),
        ],
        out_specs=[
            pl.BlockSpec((1, _BLK), lambda b, c: (0, b)),
            pl.BlockSpec((_BLK, 1), lambda b, c: (b, 0)),
        ],
        out_shape=[
            jax.ShapeDtypeStruct((1, _N), jnp.int32),
            jax.ShapeDtypeStruct((_N, 1), jnp.int32),
        ],
        scratch_shapes=[
            pltpu.VMEM((8 * _NB, _BLK), jnp.int32),
            pltpu.VMEM((_BLK, 1), jnp.int32),
        ],
    )(srec, srT)

    out = pl.pallas_call(
        _fin_kernel,
        out_shape=jax.ShapeDtypeStruct((_N, 6), jnp.float32),
    )(srec, kcol)
    return out


# speedup vs baseline: 80.9233x; 1.0434x over previous
import functools

import jax
import jax.numpy as jnp
import numpy as np
from jax import lax
from jax.experimental import pallas as pl
from jax.experimental.pallas import tpu as pltpu
from jax.experimental.pallas import tpu_sc as plsc

_NUM_CLASSES = 80
_A = 5
_H = 32
_W = 32
_N = _A * _H * _W
_HW = _H * _W
_STRIDE = 16.0
_CONF_T = 0.5
_IOU_T = 0.45
_ANCHORS = np.array(
    [[1.3221, 1.73145], [3.19275, 4.00944], [5.05587, 8.09892],
     [9.47112, 4.84053], [11.2364, 10.0071]], dtype=np.float32)
_BLK = 512
_NB = _N // _BLK
_REC = 128


def _decode_kernel(x_ref, rec_ref, srow_ref):
    pos = jax.lax.broadcasted_iota(jnp.int32, (1, _HW), 1)
    gx = (pos % _W).astype(jnp.float32)
    gy = (pos // _W).astype(jnp.float32)
    for a in range(_A):
        base = a * (5 + _NUM_CLASSES)
        blk = x_ref[base:base + 5 + _NUM_CLASSES, :]
        cx = (jax.nn.sigmoid(blk[0:1, :]) + gx) * _STRIDE
        cy = (jax.nn.sigmoid(blk[1:2, :]) + gy) * _STRIDE
        w = jnp.exp(blk[2:3, :]) * float(_ANCHORS[a, 0]) * _STRIDE
        h = jnp.exp(blk[3:4, :]) * float(_ANCHORS[a, 1]) * _STRIDE
        conf = jax.nn.sigmoid(blk[4:5, :])
        pcls = jax.nn.sigmoid(blk[5:, :])
        best = pcls[0:1, :]
        bidx = jnp.zeros((1, _HW), jnp.float32)
        for c in range(1, _NUM_CLASSES):
            cur = pcls[c:c + 1, :]
            gt = cur > best
            best = jnp.where(gt, cur, best)
            bidx = jnp.where(gt, jnp.float32(c), bidx)
        x1 = cx - w / 2.0
        y1 = cy - h / 2.0
        x2 = cx + w / 2.0
        y2 = cy + h / 2.0
        area = jnp.maximum(x2 - x1, 0.0) * jnp.maximum(y2 - y1, 0.0)
        zero = jnp.zeros((_REC - 11, _HW), jnp.float32)
        rows = jnp.concatenate(
            [cx, cy, w, h, conf, bidx, x1, y1, x2, y2, area, zero], axis=0)
        rec_ref[a * _HW:(a + 1) * _HW, :] = rows.T
        srow_ref[0:1, a * _HW:(a + 1) * _HW] = conf


def _rank_kernel(rec_ref, srow_ref, rank_ref):
    j = pl.program_id(0)
    s_col = rec_ref[:, 4:5]
    ridx = jax.lax.broadcasted_iota(jnp.int32, (_BLK, 1), 0) + j * _BLK
    acc = jnp.zeros((_BLK, 1), jnp.int32)
    for c in range(_NB):
        s_all = srow_ref[0:1, c * _BLK:(c + 1) * _BLK]
        cidx = jax.lax.broadcasted_iota(jnp.int32, (1, _BLK), 1) + c * _BLK
        ahead = (s_all > s_col) | ((s_all == s_col) & (cidx < ridx))
        acc = acc + jnp.sum(ahead.astype(jnp.int32), axis=1, keepdims=True)
    rank_ref[:, :] = acc


_SC_WORKERS = 32
_SC_PER_W = _N // _SC_WORKERS
_SC_CHUNK = 80


def _sc_scatter_body(rank_hbm, rec_hbm, out_hbm, idx_v, rows_v, sem):
    wid = lax.axis_index("s") * 2 + lax.axis_index("c")
    base = wid * _SC_PER_W
    for t in range(_SC_PER_W // _SC_CHUNK):
        off = base + t * _SC_CHUNK
        pltpu.sync_copy(rank_hbm.at[pl.ds(off, _SC_CHUNK)], idx_v.at[t])
        pltpu.sync_copy(rec_hbm.at[pl.ds(off, _SC_CHUNK)], rows_v.at[t])
        pltpu.async_copy(rows_v.at[t], out_hbm.at[idx_v.at[t]], sem).wait()


def _sc_scatter(rank_flat, rec):
    mesh = plsc.VectorSubcoreMesh(core_axis_name="c", subcore_axis_name="s")
    f = functools.partial(
        pl.kernel,
        out_type=jax.ShapeDtypeStruct((_N, _REC), jnp.float32),
        mesh=mesh,
        scratch_types=[
            pltpu.VMEM((_SC_PER_W // _SC_CHUNK, _SC_CHUNK), jnp.int32),
            pltpu.VMEM((_SC_PER_W // _SC_CHUNK, _SC_CHUNK, _REC),
                       jnp.float32),
            pltpu.SemaphoreType.DMA,
        ],
    )(_sc_scatter_body)
    return f(rank_flat, rec)


def _trans_kernel(rec_ref, srT_ref):
    srT_ref[:, :] = rec_ref[:, :].T


def _iou_tile(rec_ref, x1r, y1r, x2r, y2r, ar):
    x1c = rec_ref[:, 6:7]
    y1c = rec_ref[:, 7:8]
    x2c = rec_ref[:, 8:9]
    y2c = rec_ref[:, 9:10]
    ac = rec_ref[:, 10:11]
    xx1 = jnp.maximum(x1c, x1r)
    yy1 = jnp.maximum(y1c, y1r)
    xx2 = jnp.minimum(x2c, x2r)
    yy2 = jnp.minimum(y2c, y2r)
    inter = jnp.maximum(xx2 - xx1, 0.0) * jnp.maximum(yy2 - yy1, 0.0)
    return inter / (ac + ar - inter + 1e-9)


def _nms_kernel(rec_ref, srT_ref, kcol_ref, keff_ref, kany_ref):
    b = pl.program_id(0)
    c = pl.program_id(1)

    @pl.when((b == 0) & (c == 0))
    def _init():
        for cc in range(_NB):
            k0 = (srT_ref[4:5, cc * _BLK:(cc + 1) * _BLK]
                  > _CONF_T).astype(jnp.float32)
            keff_ref[8 * cc:8 * cc + 1, :] = k0

    @pl.when(c == b)
    def _diag():
        x1r = rec_ref[:, 6:7].T
        y1r = rec_ref[:, 7:8].T
        x2r = rec_ref[:, 8:9].T
        y2r = rec_ref[:, 9:10].T
        ar = rec_ref[:, 10:11].T
        iou = _iou_tile(rec_ref, x1r, y1r, x2r, y2r, ar)
        jl = jax.lax.broadcasted_iota(jnp.int32, (_BLK, 1), 0)
        il = jax.lax.broadcasted_iota(jnp.int32, (1, _BLK), 1)
        miou = iou * (jl < il).astype(jnp.float32)
        k0e = keff_ref[pl.ds(8 * b, 8), :][0:1, :]

        def cond(carry):
            return carry[1] > 0.0

        def body(carry):
            krow, _ = carry
            supp = jnp.max(miou * krow.T, axis=0, keepdims=True)
            knew = k0e * (1.0 - (supp > _IOU_T).astype(jnp.float32))
            chg = jnp.max(jnp.abs(knew - krow))
            return knew, chg

        kfin, _ = jax.lax.while_loop(cond, body, (k0e, jnp.float32(1.0)))
        kcol_ref[:, :] = kfin.T
        kany_ref[0, 0] = jnp.max(kfin)

    @pl.when((c > b) & (kany_ref[0, 0] > 0.0))
    def _offdiag():
        x1r = srT_ref[6:7, pl.ds(c * _BLK, _BLK)]
        y1r = srT_ref[7:8, pl.ds(c * _BLK, _BLK)]
        x2r = srT_ref[8:9, pl.ds(c * _BLK, _BLK)]
        y2r = srT_ref[9:10, pl.ds(c * _BLK, _BLK)]
        ar = srT_ref[10:11, pl.ds(c * _BLK, _BLK)]
        iou = _iou_tile(rec_ref, x1r, y1r, x2r, y2r, ar)
        supp = jnp.max(iou * kcol_ref[:, :], axis=0, keepdims=True)
        old8 = keff_ref[pl.ds(8 * c, 8), :]
        keff_ref[pl.ds(8 * c, 8), :] = old8 * (
            1.0 - (supp > _IOU_T).astype(jnp.float32))


def _fin_kernel(rec_ref, kcol_ref, out_ref):
    out_ref[:, :] = rec_ref[:, 0:6] * kcol_ref[:, :]


def kernel(x):
    X = x.reshape(_A * (5 + _NUM_CLASSES), _HW)

    rec, srow = pl.pallas_call(
        _decode_kernel,
        out_shape=[
            jax.ShapeDtypeStruct((_N, _REC), jnp.float32),
            jax.ShapeDtypeStruct((1, _N), jnp.float32),
        ],
    )(X)

    rank = pl.pallas_call(
        _rank_kernel,
        grid=(_NB,),
        in_specs=[
            pl.BlockSpec((_BLK, _REC), lambda j: (j, 0)),
            pl.BlockSpec((1, _N), lambda j: (0, 0)),
        ],
        out_specs=pl.BlockSpec((_BLK, 1), lambda j: (j, 0)),
        out_shape=jax.ShapeDtypeStruct((_N, 1), jnp.int32),
    )(rec, srow)

    srec = _sc_scatter(rank.reshape(_N), rec)

    srT = pl.pallas_call(
        _trans_kernel,
        grid=(_NB,),
        in_specs=[pl.BlockSpec((_BLK, _REC), lambda r: (r, 0))],
        out_specs=pl.BlockSpec((_REC, _BLK), lambda r: (0, r)),
        out_shape=jax.ShapeDtypeStruct((_REC, _N), jnp.float32),
    )(srec)

    kcol = pl.pallas_call(
        _nms_kernel,
        grid=(_NB, _NB),
        in_specs=[
            pl.BlockSpec((_BLK, _REC), lambda b, c: (b, 0)),
            pl.BlockSpec((_REC, _N), lambda b, c: (0, 0)),
        ],
        out_specs=pl.BlockSpec((_BLK, 1), lambda b, c: (b, 0)),
        out_shape=jax.ShapeDtypeStruct((_N, 1), jnp.float32),
        scratch_shapes=[
            pltpu.VMEM((8 * _NB, _BLK), jnp.float32),
            pltpu.SMEM((1, 1), jnp.float32),
        ],
    )(srec, srT)

    out = pl.pallas_call(
        _fin_kernel,
        out_shape=jax.ShapeDtypeStruct((_N, 6), jnp.float32),
    )(srec, kcol)
    return out


# speedup vs baseline: 94.7594x; 1.1710x over previous
import functools

import jax
import jax.numpy as jnp
import numpy as np
from jax import lax
from jax.experimental import pallas as pl
from jax.experimental.pallas import tpu as pltpu
from jax.experimental.pallas import tpu_sc as plsc

_NUM_CLASSES = 80
_A = 5
_H = 32
_W = 32
_N = _A * _H * _W
_HW = _H * _W
_STRIDE = 16.0
_CONF_T = 0.5
_IOU_T = 0.45
_ANCHORS = np.array(
    [[1.3221, 1.73145], [3.19275, 4.00944], [5.05587, 8.09892],
     [9.47112, 4.84053], [11.2364, 10.0071]], dtype=np.float32)
_BLK = 512
_NB = _N // _BLK
_REC = 128


def _decode_kernel(x_ref, rec_ref, srow_ref):
    pos = jax.lax.broadcasted_iota(jnp.int32, (1, _HW), 1)
    gx = (pos % _W).astype(jnp.float32)
    gy = (pos // _W).astype(jnp.float32)
    for a in range(_A):
        base = a * (5 + _NUM_CLASSES)
        blk = x_ref[base:base + 5 + _NUM_CLASSES, :]
        cx = (jax.nn.sigmoid(blk[0:1, :]) + gx) * _STRIDE
        cy = (jax.nn.sigmoid(blk[1:2, :]) + gy) * _STRIDE
        w = jnp.exp(blk[2:3, :]) * float(_ANCHORS[a, 0]) * _STRIDE
        h = jnp.exp(blk[3:4, :]) * float(_ANCHORS[a, 1]) * _STRIDE
        conf = jax.nn.sigmoid(blk[4:5, :])
        pcls = jax.nn.sigmoid(blk[5:, :])
        best = pcls[0:1, :]
        bidx = jnp.zeros((1, _HW), jnp.float32)
        for c in range(1, _NUM_CLASSES):
            cur = pcls[c:c + 1, :]
            gt = cur > best
            best = jnp.where(gt, cur, best)
            bidx = jnp.where(gt, jnp.float32(c), bidx)
        x1 = cx - w / 2.0
        y1 = cy - h / 2.0
        x2 = cx + w / 2.0
        y2 = cy + h / 2.0
        area = jnp.maximum(x2 - x1, 0.0) * jnp.maximum(y2 - y1, 0.0)
        zero = jnp.zeros((_REC - 11, _HW), jnp.float32)
        rows = jnp.concatenate(
            [cx, cy, w, h, conf, bidx, x1, y1, x2, y2, area, zero], axis=0)
        rec_ref[a * _HW:(a + 1) * _HW, :] = rows.T
        srow_ref[0:1, a * _HW:(a + 1) * _HW] = conf


def _rank_kernel(rec_ref, srow_ref, rank_ref):
    j = pl.program_id(0)
    s_col = rec_ref[:, 4:5]
    ridx = jax.lax.broadcasted_iota(jnp.int32, (_BLK, 1), 0) + j * _BLK
    acc = jnp.zeros((_BLK, 1), jnp.int32)
    for c in range(_NB):
        s_all = srow_ref[0:1, c * _BLK:(c + 1) * _BLK]
        cidx = jax.lax.broadcasted_iota(jnp.int32, (1, _BLK), 1) + c * _BLK
        ahead = (s_all > s_col) | ((s_all == s_col) & (cidx < ridx))
        acc = acc + jnp.sum(ahead.astype(jnp.int32), axis=1, keepdims=True)
    rank_ref[:, :] = acc


_SC_WORKERS = 32
_SC_PER_W = _N // _SC_WORKERS
_SC_CHUNK = 80


def _sc_scatter_body(rank_hbm, rec_hbm, out_hbm, idx_v, rows_v, sem):
    wid = lax.axis_index("s") * 2 + lax.axis_index("c")
    base = wid * _SC_PER_W
    for t in range(_SC_PER_W // _SC_CHUNK):
        off = base + t * _SC_CHUNK
        pltpu.sync_copy(rank_hbm.at[pl.ds(off, _SC_CHUNK)], idx_v.at[t])
        pltpu.sync_copy(rec_hbm.at[pl.ds(off, _SC_CHUNK)], rows_v.at[t])
        pltpu.async_copy(rows_v.at[t], out_hbm.at[idx_v.at[t]], sem).wait()


def _sc_scatter(rank_flat, rec):
    mesh = plsc.VectorSubcoreMesh(core_axis_name="c", subcore_axis_name="s")
    f = functools.partial(
        pl.kernel,
        out_type=jax.ShapeDtypeStruct((_N, _REC), jnp.float32),
        mesh=mesh,
        scratch_types=[
            pltpu.VMEM((_SC_PER_W // _SC_CHUNK, _SC_CHUNK), jnp.int32),
            pltpu.VMEM((_SC_PER_W // _SC_CHUNK, _SC_CHUNK, _REC),
                       jnp.float32),
            pltpu.SemaphoreType.DMA,
        ],
    )(_sc_scatter_body)
    return f(rank_flat, rec)


_STARTS = [b * _NB - b * (b - 1) // 2 for b in range(_NB)]


def _b_of(i):
    b = jnp.int32(0)
    for k in range(1, _NB):
        b = b + (i >= _STARTS[k]).astype(jnp.int32)
    return b


def _bc_of(i):
    b = _b_of(i)
    start = b * _NB - (b * (b - 1)) // 2
    return b, i - start + b


def _iou_tile(rec_ref, x1r, y1r, x2r, y2r, ar):
    x1c = rec_ref[:, 6:7]
    y1c = rec_ref[:, 7:8]
    x2c = rec_ref[:, 8:9]
    y2c = rec_ref[:, 9:10]
    ac = rec_ref[:, 10:11]
    xx1 = jnp.maximum(x1c, x1r)
    yy1 = jnp.maximum(y1c, y1r)
    xx2 = jnp.minimum(x2c, x2r)
    yy2 = jnp.minimum(y2c, y2r)
    inter = jnp.maximum(xx2 - xx1, 0.0) * jnp.maximum(yy2 - yy1, 0.0)
    return inter / (ac + ar - inter + 1e-9)


def _nms_kernel(rec_ref, out_ref, srT_ref, kcol_ref, keff_ref, kany_ref):
    b, c = _bc_of(pl.program_id(0))

    @pl.when(pl.program_id(0) == 0)
    def _init():
        for cc in range(_NB):
            blk = rec_ref[cc * _BLK:(cc + 1) * _BLK, :]
            srT_ref[:, cc * _BLK:(cc + 1) * _BLK] = blk.T
        for cc in range(_NB):
            k0 = (srT_ref[4:5, cc * _BLK:(cc + 1) * _BLK]
                  > _CONF_T).astype(jnp.float32)
            keff_ref[8 * cc:8 * cc + 1, :] = k0

    recb = rec_ref[pl.ds(b * _BLK, _BLK), :]

    @pl.when(c == b)
    def _diag():
        x1r = srT_ref[6:7, pl.ds(b * _BLK, _BLK)]
        y1r = srT_ref[7:8, pl.ds(b * _BLK, _BLK)]
        x2r = srT_ref[8:9, pl.ds(b * _BLK, _BLK)]
        y2r = srT_ref[9:10, pl.ds(b * _BLK, _BLK)]
        ar = srT_ref[10:11, pl.ds(b * _BLK, _BLK)]
        iou = _iou_tile(recb, x1r, y1r, x2r, y2r, ar)
        jl = jax.lax.broadcasted_iota(jnp.int32, (_BLK, 1), 0)
        il = jax.lax.broadcasted_iota(jnp.int32, (1, _BLK), 1)
        miou = iou * (jl < il).astype(jnp.float32)
        k0e = keff_ref[pl.ds(8 * b, 8), :][0:1, :]

        def cond(carry):
            return carry[1] > 0.0

        def body(carry):
            krow, _ = carry
            supp = jnp.max(miou * krow.T, axis=0, keepdims=True)
            knew = k0e * (1.0 - (supp > _IOU_T).astype(jnp.float32))
            chg = jnp.max(jnp.abs(knew - krow))
            return knew, chg

        kfin, _ = jax.lax.while_loop(cond, body, (k0e, jnp.float32(1.0)))
        kcol_ref[:, :] = kfin.T
        kany_ref[0, 0] = jnp.max(kfin)

    @pl.when((c > b) & (kany_ref[0, 0] > 0.0))
    def _offdiag():
        x1r = srT_ref[6:7, pl.ds(c * _BLK, _BLK)]
        y1r = srT_ref[7:8, pl.ds(c * _BLK, _BLK)]
        x2r = srT_ref[8:9, pl.ds(c * _BLK, _BLK)]
        y2r = srT_ref[9:10, pl.ds(c * _BLK, _BLK)]
        ar = srT_ref[10:11, pl.ds(c * _BLK, _BLK)]
        iou = _iou_tile(recb, x1r, y1r, x2r, y2r, ar)
        supp = jnp.max(iou * kcol_ref[:, :], axis=0, keepdims=True)
        old8 = keff_ref[pl.ds(8 * c, 8), :]
        keff_ref[pl.ds(8 * c, 8), :] = old8 * (
            1.0 - (supp > _IOU_T).astype(jnp.float32))

    @pl.when(c == _NB - 1)
    def _emit():
        out_ref[:, :] = recb[:, 0:6] * kcol_ref[:, :]


def kernel(x):
    X = x.reshape(_A * (5 + _NUM_CLASSES), _HW)

    rec, srow = pl.pallas_call(
        _decode_kernel,
        out_shape=[
            jax.ShapeDtypeStruct((_N, _REC), jnp.float32),
            jax.ShapeDtypeStruct((1, _N), jnp.float32),
        ],
    )(X)

    rank = pl.pallas_call(
        _rank_kernel,
        grid=(_NB,),
        in_specs=[
            pl.BlockSpec((_BLK, _REC), lambda j: (j, 0)),
            pl.BlockSpec((1, _N), lambda j: (0, 0)),
        ],
        out_specs=pl.BlockSpec((_BLK, 1), lambda j: (j, 0)),
        out_shape=jax.ShapeDtypeStruct((_N, 1), jnp.int32),
    )(rec, srow)

    srec = _sc_scatter(rank.reshape(_N), rec)

    out = pl.pallas_call(
        _nms_kernel,
        grid=(_STARTS[-1] + 1,),
        in_specs=[pl.BlockSpec((_N, _REC), lambda i: (0, 0))],
        out_specs=pl.BlockSpec((_BLK, 6), lambda i: (_b_of(i), 0)),
        out_shape=jax.ShapeDtypeStruct((_N, 6), jnp.float32),
        scratch_shapes=[
            pltpu.VMEM((_REC, _N), jnp.float32),
            pltpu.VMEM((_BLK, 1), jnp.float32),
            pltpu.VMEM((8 * _NB, _BLK), jnp.float32),
            pltpu.SMEM((1, 1), jnp.float32),
        ],
    )(srec)
    return out


# speedup vs baseline: 111.9676x; 1.1816x over previous
import functools

import jax
import jax.numpy as jnp
import numpy as np
from jax import lax
from jax.experimental import pallas as pl
from jax.experimental.pallas import tpu as pltpu
from jax.experimental.pallas import tpu_sc as plsc

_NUM_CLASSES = 80
_A = 5
_H = 32
_W = 32
_N = _A * _H * _W
_HW = _H * _W
_STRIDE = 16.0
_CONF_T = 0.5
_IOU_T = 0.45
_ANCHORS = np.array(
    [[1.3221, 1.73145], [3.19275, 4.00944], [5.05587, 8.09892],
     [9.47112, 4.84053], [11.2364, 10.0071]], dtype=np.float32)
_BLK = 512
_NB = _N // _BLK
_REC = 128


def _decode_kernel(x_ref, rec_ref, srow_ref):
    pos = jax.lax.broadcasted_iota(jnp.int32, (1, _HW), 1)
    gx = (pos % _W).astype(jnp.float32)
    gy = (pos // _W).astype(jnp.float32)
    for a in range(_A):
        base = a * (5 + _NUM_CLASSES)
        blk = x_ref[base:base + 5 + _NUM_CLASSES, :]
        cx = (jax.nn.sigmoid(blk[0:1, :]) + gx) * _STRIDE
        cy = (jax.nn.sigmoid(blk[1:2, :]) + gy) * _STRIDE
        w = jnp.exp(blk[2:3, :]) * float(_ANCHORS[a, 0]) * _STRIDE
        h = jnp.exp(blk[3:4, :]) * float(_ANCHORS[a, 1]) * _STRIDE
        conf = jax.nn.sigmoid(blk[4:5, :])
        pcls = jax.nn.sigmoid(blk[5:, :])
        best = pcls[0:1, :]
        bidx = jnp.zeros((1, _HW), jnp.float32)
        for c in range(1, _NUM_CLASSES):
            cur = pcls[c:c + 1, :]
            gt = cur > best
            best = jnp.where(gt, cur, best)
            bidx = jnp.where(gt, jnp.float32(c), bidx)
        x1 = cx - w / 2.0
        y1 = cy - h / 2.0
        x2 = cx + w / 2.0
        y2 = cy + h / 2.0
        area = jnp.maximum(x2 - x1, 0.0) * jnp.maximum(y2 - y1, 0.0)
        zero = jnp.zeros((_REC - 11, _HW), jnp.float32)
        rows = jnp.concatenate(
            [cx, cy, w, h, conf, bidx, x1, y1, x2, y2, area, zero], axis=0)
        rec_ref[a * _HW:(a + 1) * _HW, :] = rows.T
        srow_ref[0:1, a * _HW:(a + 1) * _HW] = conf


def _rank_kernel(rec_ref, srow_ref, rank_ref):
    j = pl.program_id(0)
    s_col = rec_ref[:, 4:5]
    ridx = jax.lax.broadcasted_iota(jnp.int32, (_BLK, 1), 0) + j * _BLK
    acc = jnp.zeros((_BLK, 1), jnp.int32)
    for c in range(_NB):
        s_all = srow_ref[0:1, c * _BLK:(c + 1) * _BLK]
        cidx = jax.lax.broadcasted_iota(jnp.int32, (1, _BLK), 1) + c * _BLK
        ahead = (s_all > s_col) | ((s_all == s_col) & (cidx < ridx))
        acc = acc + jnp.sum(ahead.astype(jnp.int32), axis=1, keepdims=True)
    rank_ref[:, :] = acc


_SC_WORKERS = 32
_SC_PER_W = _N // _SC_WORKERS
_SC_CHUNK = 80


def _sc_scatter_body(rank_hbm, rec_hbm, out_hbm, idx_v, rows_v, sem):
    wid = lax.axis_index("s") * 2 + lax.axis_index("c")
    base = wid * _SC_PER_W
    for t in range(_SC_PER_W // _SC_CHUNK):
        off = base + t * _SC_CHUNK
        pltpu.sync_copy(rank_hbm.at[pl.ds(off, _SC_CHUNK)], idx_v.at[t])
        pltpu.sync_copy(rec_hbm.at[pl.ds(off, _SC_CHUNK)], rows_v.at[t])
        pltpu.async_copy(rows_v.at[t], out_hbm.at[idx_v.at[t]], sem).wait()


def _sc_scatter(rank_flat, rec):
    mesh = plsc.VectorSubcoreMesh(core_axis_name="c", subcore_axis_name="s")
    f = functools.partial(
        pl.kernel,
        out_type=jax.ShapeDtypeStruct((_N, _REC), jnp.float32),
        mesh=mesh,
        scratch_types=[
            pltpu.VMEM((_SC_PER_W // _SC_CHUNK, _SC_CHUNK), jnp.int32),
            pltpu.VMEM((_SC_PER_W // _SC_CHUNK, _SC_CHUNK, _REC),
                       jnp.float32),
            pltpu.SemaphoreType.DMA,
        ],
    )(_sc_scatter_body)
    return f(rank_flat, rec)


_STARTS = [b * _NB - b * (b - 1) // 2 for b in range(_NB)]


def _b_of(i):
    b = jnp.int32(0)
    for k in range(1, _NB):
        b = b + (i >= _STARTS[k]).astype(jnp.int32)
    return b


def _bc_of(i):
    b = _b_of(i)
    start = b * _NB - (b * (b - 1)) // 2
    return b, i - start + b


def _iou_tile(rec_ref, x1r, y1r, x2r, y2r, ar):
    x1c = rec_ref[:, 6:7]
    y1c = rec_ref[:, 7:8]
    x2c = rec_ref[:, 8:9]
    y2c = rec_ref[:, 9:10]
    ac = rec_ref[:, 10:11]
    xx1 = jnp.maximum(x1c, x1r)
    yy1 = jnp.maximum(y1c, y1r)
    xx2 = jnp.minimum(x2c, x2r)
    yy2 = jnp.minimum(y2c, y2r)
    inter = jnp.maximum(xx2 - xx1, 0.0) * jnp.maximum(yy2 - yy1, 0.0)
    return inter / (ac + ar - inter + 1e-9)


def _nms_kernel(rec_ref, out_ref, srT_ref, kcol_ref, keff_ref, kany_ref):
    b, c = _bc_of(pl.program_id(0))

    @pl.when(pl.program_id(0) == 0)
    def _init():
        for cc in range(_NB):
            blk = rec_ref[cc * _BLK:(cc + 1) * _BLK, :]
            srT_ref[:, cc * _BLK:(cc + 1) * _BLK] = blk.T
        for cc in range(_NB):
            k0 = (srT_ref[4:5, cc * _BLK:(cc + 1) * _BLK]
                  > _CONF_T).astype(jnp.float32)
            keff_ref[8 * cc:8 * cc + 1, :] = k0

    recb = rec_ref[pl.ds(b * _BLK, _BLK), :]

    @pl.when(c == b)
    def _diag():
        k0e = keff_ref[pl.ds(8 * b, 8), :][0:1, :]
        kcol_ref[:, :] = jnp.zeros((_BLK, 1), jnp.float32)
        kany_ref[0, 0] = jnp.max(k0e)

        @pl.when(kany_ref[0, 0] > 0.0)
        def _active():
            x1r = srT_ref[6:7, pl.ds(b * _BLK, _BLK)]
            y1r = srT_ref[7:8, pl.ds(b * _BLK, _BLK)]
            x2r = srT_ref[8:9, pl.ds(b * _BLK, _BLK)]
            y2r = srT_ref[9:10, pl.ds(b * _BLK, _BLK)]
            ar = srT_ref[10:11, pl.ds(b * _BLK, _BLK)]
            iou = _iou_tile(recb, x1r, y1r, x2r, y2r, ar)
            jl = jax.lax.broadcasted_iota(jnp.int32, (_BLK, 1), 0)
            il = jax.lax.broadcasted_iota(jnp.int32, (1, _BLK), 1)
            miou = iou * (jl < il).astype(jnp.float32)

            def cond(carry):
                return carry[1] > 0.0

            def body(carry):
                krow, _ = carry
                supp = jnp.max(miou * krow.T, axis=0, keepdims=True)
                knew = k0e * (1.0 - (supp > _IOU_T).astype(jnp.float32))
                chg = jnp.max(jnp.abs(knew - krow))
                return knew, chg

            kfin, _ = jax.lax.while_loop(cond, body, (k0e, jnp.float32(1.0)))
            kcol_ref[:, :] = kfin.T

    old8 = keff_ref[pl.ds(8 * c, 8), :]
    tgt_any = jnp.max(old8[0:1, :])

    @pl.when((c > b) & (kany_ref[0, 0] > 0.0) & (tgt_any > 0.0))
    def _offdiag():
        x1r = srT_ref[6:7, pl.ds(c * _BLK, _BLK)]
        y1r = srT_ref[7:8, pl.ds(c * _BLK, _BLK)]
        x2r = srT_ref[8:9, pl.ds(c * _BLK, _BLK)]
        y2r = srT_ref[9:10, pl.ds(c * _BLK, _BLK)]
        ar = srT_ref[10:11, pl.ds(c * _BLK, _BLK)]
        iou = _iou_tile(recb, x1r, y1r, x2r, y2r, ar)
        supp = jnp.max(iou * kcol_ref[:, :], axis=0, keepdims=True)
        keff_ref[pl.ds(8 * c, 8), :] = old8 * (
            1.0 - (supp > _IOU_T).astype(jnp.float32))

    @pl.when(c == _NB - 1)
    def _emit():
        out_ref[:, :] = recb[:, 0:6] * kcol_ref[:, :]


def kernel(x):
    X = x.reshape(_A * (5 + _NUM_CLASSES), _HW)

    rec, srow = pl.pallas_call(
        _decode_kernel,
        out_shape=[
            jax.ShapeDtypeStruct((_N, _REC), jnp.float32),
            jax.ShapeDtypeStruct((1, _N), jnp.float32),
        ],
    )(X)

    rank = pl.pallas_call(
        _rank_kernel,
        grid=(_NB,),
        in_specs=[
            pl.BlockSpec((_BLK, _REC), lambda j: (j, 0)),
            pl.BlockSpec((1, _N), lambda j: (0, 0)),
        ],
        out_specs=pl.BlockSpec((_BLK, 1), lambda j: (j, 0)),
        out_shape=jax.ShapeDtypeStruct((_N, 1), jnp.int32),
    )(rec, srow)

    srec = _sc_scatter(rank.reshape(_N), rec)

    out = pl.pallas_call(
        _nms_kernel,
        grid=(_STARTS[-1] + 1,),
        in_specs=[pl.BlockSpec((_N, _REC), lambda i: (0, 0))],
        out_specs=pl.BlockSpec((_BLK, 6), lambda i: (_b_of(i), 0)),
        out_shape=jax.ShapeDtypeStruct((_N, 6), jnp.float32),
        scratch_shapes=[
            pltpu.VMEM((_REC, _N), jnp.float32),
            pltpu.VMEM((_BLK, 1), jnp.float32),
            pltpu.VMEM((8 * _NB, _BLK), jnp.float32),
            pltpu.SMEM((1, 1), jnp.float32),
        ],
    )(srec)
    return out


# speedup vs baseline: 124.2588x; 1.1098x over previous
import functools

import jax
import jax.numpy as jnp
import numpy as np
from jax import lax
from jax.experimental import pallas as pl
from jax.experimental.pallas import tpu as pltpu
from jax.experimental.pallas import tpu_sc as plsc

_NUM_CLASSES = 80
_A = 5
_H = 32
_W = 32
_N = _A * _H * _W
_HW = _H * _W
_STRIDE = 16.0
_CONF_T = 0.5
_IOU_T = 0.45
_ANCHORS = np.array(
    [[1.3221, 1.73145], [3.19275, 4.00944], [5.05587, 8.09892],
     [9.47112, 4.84053], [11.2364, 10.0071]], dtype=np.float32)
_BLK = 1024
_NB = _N // _BLK
_REC = 128


def _decode_kernel(x_ref, rec_ref, srow_ref):
    pos = jax.lax.broadcasted_iota(jnp.int32, (1, _HW), 1)
    gx = (pos % _W).astype(jnp.float32)
    gy = (pos // _W).astype(jnp.float32)
    for a in range(_A):
        base = a * (5 + _NUM_CLASSES)
        blk = x_ref[base:base + 5 + _NUM_CLASSES, :]
        cx = (jax.nn.sigmoid(blk[0:1, :]) + gx) * _STRIDE
        cy = (jax.nn.sigmoid(blk[1:2, :]) + gy) * _STRIDE
        w = jnp.exp(blk[2:3, :]) * float(_ANCHORS[a, 0]) * _STRIDE
        h = jnp.exp(blk[3:4, :]) * float(_ANCHORS[a, 1]) * _STRIDE
        conf = jax.nn.sigmoid(blk[4:5, :])
        pcls = jax.nn.sigmoid(blk[5:, :])
        best = pcls[0:1, :]
        bidx = jnp.zeros((1, _HW), jnp.float32)
        for c in range(1, _NUM_CLASSES):
            cur = pcls[c:c + 1, :]
            gt = cur > best
            best = jnp.where(gt, cur, best)
            bidx = jnp.where(gt, jnp.float32(c), bidx)
        x1 = cx - w / 2.0
        y1 = cy - h / 2.0
        x2 = cx + w / 2.0
        y2 = cy + h / 2.0
        area = jnp.maximum(x2 - x1, 0.0) * jnp.maximum(y2 - y1, 0.0)
        zero = jnp.zeros((_REC - 11, _HW), jnp.float32)
        rows = jnp.concatenate(
            [cx, cy, w, h, conf, bidx, x1, y1, x2, y2, area, zero], axis=0)
        rec_ref[a * _HW:(a + 1) * _HW, :] = rows.T
        srow_ref[0:1, a * _HW:(a + 1) * _HW] = conf


def _rank_kernel(rec_ref, srow_ref, rank_ref):
    j = pl.program_id(0)
    s_col = rec_ref[:, 4:5]
    ridx = jax.lax.broadcasted_iota(jnp.int32, (_BLK, 1), 0) + j * _BLK
    acc = jnp.zeros((_BLK, 1), jnp.int32)
    for c in range(_NB):
        s_all = srow_ref[0:1, c * _BLK:(c + 1) * _BLK]
        cidx = jax.lax.broadcasted_iota(jnp.int32, (1, _BLK), 1) + c * _BLK
        ahead = (s_all > s_col) | ((s_all == s_col) & (cidx < ridx))
        acc = acc + jnp.sum(ahead.astype(jnp.int32), axis=1, keepdims=True)
    rank_ref[:, :] = acc


_SC_WORKERS = 32
_SC_PER_W = _N // _SC_WORKERS
_SC_CHUNK = 80


def _sc_scatter_body(rank_hbm, rec_hbm, out_hbm, idx_v, rows_v, sem):
    wid = lax.axis_index("s") * 2 + lax.axis_index("c")
    base = wid * _SC_PER_W
    for t in range(_SC_PER_W // _SC_CHUNK):
        off = base + t * _SC_CHUNK
        pltpu.sync_copy(rank_hbm.at[pl.ds(off, _SC_CHUNK)], idx_v.at[t])
        pltpu.sync_copy(rec_hbm.at[pl.ds(off, _SC_CHUNK)], rows_v.at[t])
        pltpu.async_copy(rows_v.at[t], out_hbm.at[idx_v.at[t]], sem).wait()


def _sc_scatter(rank_flat, rec):
    mesh = plsc.VectorSubcoreMesh(core_axis_name="c", subcore_axis_name="s")
    f = functools.partial(
        pl.kernel,
        out_type=jax.ShapeDtypeStruct((_N, _REC), jnp.float32),
        mesh=mesh,
        scratch_types=[
            pltpu.VMEM((_SC_PER_W // _SC_CHUNK, _SC_CHUNK), jnp.int32),
            pltpu.VMEM((_SC_PER_W // _SC_CHUNK, _SC_CHUNK, _REC),
                       jnp.float32),
            pltpu.SemaphoreType.DMA,
        ],
    )(_sc_scatter_body)
    return f(rank_flat, rec)


_STARTS = [b * _NB - b * (b - 1) // 2 for b in range(_NB)]


def _b_of(i):
    b = jnp.int32(0)
    for k in range(1, _NB):
        b = b + (i >= _STARTS[k]).astype(jnp.int32)
    return b


def _bc_of(i):
    b = _b_of(i)
    start = b * _NB - (b * (b - 1)) // 2
    return b, i - start + b


def _iou_tile(rec_ref, x1r, y1r, x2r, y2r, ar):
    x1c = rec_ref[:, 6:7]
    y1c = rec_ref[:, 7:8]
    x2c = rec_ref[:, 8:9]
    y2c = rec_ref[:, 9:10]
    ac = rec_ref[:, 10:11]
    xx1 = jnp.maximum(x1c, x1r)
    yy1 = jnp.maximum(y1c, y1r)
    xx2 = jnp.minimum(x2c, x2r)
    yy2 = jnp.minimum(y2c, y2r)
    inter = jnp.maximum(xx2 - xx1, 0.0) * jnp.maximum(yy2 - yy1, 0.0)
    return inter / (ac + ar - inter + 1e-9)


def _nms_kernel(rec_ref, out_ref, srT_ref, kcol_ref, keff_ref, kany_ref):
    b, c = _bc_of(pl.program_id(0))

    @pl.when(pl.program_id(0) == 0)
    def _init():
        for cc in range(_NB):
            blk = rec_ref[cc * _BLK:(cc + 1) * _BLK, :]
            srT_ref[:, cc * _BLK:(cc + 1) * _BLK] = blk.T
        for cc in range(_NB):
            k0 = (srT_ref[4:5, cc * _BLK:(cc + 1) * _BLK]
                  > _CONF_T).astype(jnp.float32)
            keff_ref[8 * cc:8 * cc + 1, :] = k0

    recb = rec_ref[pl.ds(b * _BLK, _BLK), :]

    @pl.when(c == b)
    def _diag():
        k0e = keff_ref[pl.ds(8 * b, 8), :][0:1, :]
        kcol_ref[:, :] = jnp.zeros((_BLK, 1), jnp.float32)
        kany_ref[0, 0] = jnp.max(k0e)

        @pl.when(kany_ref[0, 0] > 0.0)
        def _active():
            x1r = srT_ref[6:7, pl.ds(b * _BLK, _BLK)]
            y1r = srT_ref[7:8, pl.ds(b * _BLK, _BLK)]
            x2r = srT_ref[8:9, pl.ds(b * _BLK, _BLK)]
            y2r = srT_ref[9:10, pl.ds(b * _BLK, _BLK)]
            ar = srT_ref[10:11, pl.ds(b * _BLK, _BLK)]
            iou = _iou_tile(recb, x1r, y1r, x2r, y2r, ar)
            jl = jax.lax.broadcasted_iota(jnp.int32, (_BLK, 1), 0)
            il = jax.lax.broadcasted_iota(jnp.int32, (1, _BLK), 1)
            miou = iou * (jl < il).astype(jnp.float32)

            def cond(carry):
                return carry[1] > 0.0

            def body(carry):
                krow, _ = carry
                supp = jnp.max(miou * krow.T, axis=0, keepdims=True)
                knew = k0e * (1.0 - (supp > _IOU_T).astype(jnp.float32))
                chg = jnp.max(jnp.abs(knew - krow))
                return knew, chg

            kfin, _ = jax.lax.while_loop(cond, body, (k0e, jnp.float32(1.0)))
            kcol_ref[:, :] = kfin.T

    old8 = keff_ref[pl.ds(8 * c, 8), :]
    tgt_any = jnp.max(old8[0:1, :])

    @pl.when((c > b) & (kany_ref[0, 0] > 0.0) & (tgt_any > 0.0))
    def _offdiag():
        x1r = srT_ref[6:7, pl.ds(c * _BLK, _BLK)]
        y1r = srT_ref[7:8, pl.ds(c * _BLK, _BLK)]
        x2r = srT_ref[8:9, pl.ds(c * _BLK, _BLK)]
        y2r = srT_ref[9:10, pl.ds(c * _BLK, _BLK)]
        ar = srT_ref[10:11, pl.ds(c * _BLK, _BLK)]
        iou = _iou_tile(recb, x1r, y1r, x2r, y2r, ar)
        supp = jnp.max(iou * kcol_ref[:, :], axis=0, keepdims=True)
        keff_ref[pl.ds(8 * c, 8), :] = old8 * (
            1.0 - (supp > _IOU_T).astype(jnp.float32))

    @pl.when(c == _NB - 1)
    def _emit():
        out_ref[:, :] = recb[:, 0:6] * kcol_ref[:, :]


def kernel(x):
    X = x.reshape(_A * (5 + _NUM_CLASSES), _HW)

    rec, srow = pl.pallas_call(
        _decode_kernel,
        out_shape=[
            jax.ShapeDtypeStruct((_N, _REC), jnp.float32),
            jax.ShapeDtypeStruct((1, _N), jnp.float32),
        ],
    )(X)

    rank = pl.pallas_call(
        _rank_kernel,
        grid=(_NB,),
        in_specs=[
            pl.BlockSpec((_BLK, _REC), lambda j: (j, 0)),
            pl.BlockSpec((1, _N), lambda j: (0, 0)),
        ],
        out_specs=pl.BlockSpec((_BLK, 1), lambda j: (j, 0)),
        out_shape=jax.ShapeDtypeStruct((_N, 1), jnp.int32),
    )(rec, srow)

    srec = _sc_scatter(rank.reshape(_N), rec)

    out = pl.pallas_call(
        _nms_kernel,
        grid=(_STARTS[-1] + 1,),
        in_specs=[pl.BlockSpec((_N, _REC), lambda i: (0, 0))],
        out_specs=pl.BlockSpec((_BLK, 6), lambda i: (_b_of(i), 0)),
        out_shape=jax.ShapeDtypeStruct((_N, 6), jnp.float32),
        scratch_shapes=[
            pltpu.VMEM((_REC, _N), jnp.float32),
            pltpu.VMEM((_BLK, 1), jnp.float32),
            pltpu.VMEM((8 * _NB, _BLK), jnp.float32),
            pltpu.SMEM((1, 1), jnp.float32),
        ],
    )(srec)
    return out
